# baseline (speedup 1.0000x reference)
import functools

import jax
import jax.numpy as jnp
from jax import lax
from jax.experimental import pallas as pl
from jax.experimental.pallas import tpu as pltpu
from jax.experimental.pallas import tpu_sc as plsc

EMB = 128
NC, NS = 2, 16
NW = NC * NS
IDX_CHUNK = 128


def _gather_embeddings(user_emb, item_emb, uid2, iid2):
    B = uid2.shape[0] * uid2.shape[1]
    b_per_w = B // NW
    n_chunks = b_per_w // IDX_CHUNK

    mesh = plsc.VectorSubcoreMesh(core_axis_name="c", subcore_axis_name="s")

    n_total = 2 * n_chunks
    NBUF = 4

    @functools.partial(
        pl.kernel,
        mesh=mesh,
        out_type=[
            jax.ShapeDtypeStruct((B, EMB), jnp.float32),
            jax.ShapeDtypeStruct((B, EMB), jnp.float32),
        ],
        scratch_types=[
            pltpu.VMEM((n_total, IDX_CHUNK), jnp.int32),
            pltpu.VMEM((NBUF * IDX_CHUNK, EMB), jnp.float32),
            pltpu.SemaphoreType.DMA,
            pltpu.SemaphoreType.DMA,
        ],
    )
    def gather(ut_hbm, it_hbm, uid_hbm, iid_hbm, ue_hbm, ie_hbm,
               idx_v, bufs, sem_g, sem_w):
        wid = lax.axis_index("s") * NC + lax.axis_index("c")
        base = wid * b_per_w
        pltpu.sync_copy(uid_hbm.at[pl.ds(wid * n_chunks, n_chunks)],
                        idx_v.at[pl.ds(0, n_chunks)])
        pltpu.sync_copy(iid_hbm.at[pl.ds(wid * n_chunks, n_chunks)],
                        idx_v.at[pl.ds(n_chunks, n_chunks)])

        def fire_g(c):
            tab = ut_hbm if c < n_chunks else it_hbm
            return pltpu.async_copy(
                tab.at[idx_v.at[c]],
                bufs.at[pl.ds((c % NBUF) * IDX_CHUNK, IDX_CHUNK)], sem_g)

        def fire_w(c):
            out = ue_hbm if c < n_chunks else ie_hbm
            off = base + (c % n_chunks) * IDX_CHUNK
            return pltpu.async_copy(
                bufs.at[pl.ds((c % NBUF) * IDX_CHUNK, IDX_CHUNK)],
                out.at[pl.ds(off, IDX_CHUNK)], sem_w)

        gd = [None] * n_total
        wd = [None] * n_total
        for c in range(NBUF):
            gd[c] = fire_g(c)
        for c in range(n_total):
            gd[c].wait()
            wd[c] = fire_w(c)
            if c + NBUF < n_total:
                wd[c].wait()
                gd[c + NBUF] = fire_g(c + NBUF)
        for c in range(n_total - NBUF, n_total):
            wd[c].wait()

    return gather(user_emb, item_emb, uid2, iid2)


def _ln_mxu(h, g, b, eps=1e-5):
    n = h.shape[-1]
    avg = jnp.full((n, n), 1.0 / n, jnp.bfloat16)
    mu = jnp.dot(h.astype(jnp.bfloat16), avg, preferred_element_type=jnp.float32)
    d = h - mu
    var = jnp.dot((d * d).astype(jnp.bfloat16), avg,
                  preferred_element_type=jnp.float32)
    return d * jax.lax.rsqrt(var + eps) * g + b


def _mlp_body(ue_ref, ie_ref, w1u_ref, w1i_ref, b1_ref, g1_ref, be1_ref,
              w2_ref, b2_ref, g2_ref, be2_ref, w3_ref, b3_ref,
              w4_ref, b4_ref, w5_ref, b5_ref, bias_ref, out_ref):
    ue = ue_ref[...].astype(jnp.bfloat16)
    ie = ie_ref[...].astype(jnp.bfloat16)
    ones_col = jnp.full((EMB, 1), 1.0, jnp.bfloat16)
    fm = jnp.dot(ue * ie, ones_col, preferred_element_type=jnp.float32)
    h = jnp.dot(ue, w1u_ref[...], preferred_element_type=jnp.float32)
    h = h + jnp.dot(ie, w1i_ref[...], preferred_element_type=jnp.float32)
    h = jnp.maximum(h + b1_ref[...], 0.0)
    h = _ln_mxu(h, g1_ref[...], be1_ref[...])
    h = jnp.dot(h.astype(jnp.bfloat16), w2_ref[...],
                preferred_element_type=jnp.float32)
    h = jnp.maximum(h + b2_ref[...], 0.0)
    h = _ln_mxu(h, g2_ref[...], be2_ref[...])
    h = jnp.dot(h.astype(jnp.bfloat16), w3_ref[...],
                preferred_element_type=jnp.float32)
    h = jnp.maximum(h + b3_ref[...], 0.0)
    h = jnp.dot(h.astype(jnp.bfloat16), w4_ref[...],
                preferred_element_type=jnp.float32)
    h = jnp.maximum(h + b4_ref[...], 0.0)
    dnn = jnp.dot(h.astype(jnp.bfloat16), w5_ref[...],
                  preferred_element_type=jnp.float32) + b5_ref[...]
    t = fm + dnn + bias_ref[...]
    t = t.reshape(t.shape[0] // 128, 128)
    out_ref[...] = jax.nn.sigmoid(t)


def _mlp(ue, ie, *params, block_b=4096):
    B = ue.shape[0]
    grid = (B // block_b,)

    def _full(p):
        return pl.BlockSpec(p.shape, lambda i: (0,) * p.ndim)

    in_specs = [
        pl.BlockSpec((block_b, EMB), lambda i: (i, 0)),
        pl.BlockSpec((block_b, EMB), lambda i: (i, 0)),
    ] + [_full(p) for p in params]

    return pl.pallas_call(
        _mlp_body,
        grid=grid,
        in_specs=in_specs,
        out_specs=pl.BlockSpec((block_b // 128, 128), lambda i: (i, 0)),
        out_shape=jax.ShapeDtypeStruct((B // 128, 128), jnp.float32),
        compiler_params=pltpu.CompilerParams(
            dimension_semantics=("parallel",),
        ),
    )(ue, ie, *params)


def kernel(x, user_emb, item_emb, bias, W1, b1, g1, be1, W2, b2, g2, be2,
           W3, b3, W4, b4, W5, b5):
    uid2 = x[:, 0].astype(jnp.int32).reshape(-1, IDX_CHUNK)
    iid2 = x[:, 1].astype(jnp.int32).reshape(-1, IDX_CHUNK)
    params = (
        W1[:, :EMB].T.astype(jnp.bfloat16), W1[:, EMB:].T.astype(jnp.bfloat16),
        b1.reshape(1, -1), g1.reshape(1, -1), be1.reshape(1, -1),
        W2.T.astype(jnp.bfloat16), b2.reshape(1, -1), g2.reshape(1, -1),
        be2.reshape(1, -1),
        W3.T.astype(jnp.bfloat16), b3.reshape(1, -1),
        W4.T.astype(jnp.bfloat16), b4.reshape(1, -1),
        W5.T.astype(jnp.bfloat16), b5.reshape(1, -1),
        bias.reshape(1, 1),
    )
    ue, ie = _gather_embeddings(user_emb, item_emb, uid2, iid2)
    return _mlp(ue, ie, *params).reshape(-1)


# speedup vs baseline: 1.0446x; 1.0446x over previous
import functools

import jax
import jax.numpy as jnp
from jax import lax
from jax.experimental import pallas as pl
from jax.experimental.pallas import tpu as pltpu
from jax.experimental.pallas import tpu_sc as plsc

EMB = 128
NC, NS = 2, 16
NW = NC * NS
IDX_CHUNK = 128


def _gather_embeddings(user_emb, item_emb, uid2, iid2):
    B = uid2.shape[0] * uid2.shape[1]
    b_per_w = B // NW
    n_chunks = b_per_w // IDX_CHUNK

    mesh = plsc.VectorSubcoreMesh(core_axis_name="c", subcore_axis_name="s")

    n_total = 2 * n_chunks
    NBUF = 4

    @functools.partial(
        pl.kernel,
        mesh=mesh,
        out_type=[
            jax.ShapeDtypeStruct((B, EMB), jnp.float32),
            jax.ShapeDtypeStruct((B, EMB), jnp.float32),
        ],
        scratch_types=[
            pltpu.VMEM((n_total, IDX_CHUNK), jnp.int32),
            pltpu.VMEM((NBUF * IDX_CHUNK, EMB), jnp.float32),
            pltpu.SemaphoreType.DMA,
            pltpu.SemaphoreType.DMA,
        ],
    )
    def gather(ut_hbm, it_hbm, uid_hbm, iid_hbm, ue_hbm, ie_hbm,
               idx_v, bufs, sem_g, sem_w):
        wid = lax.axis_index("s") * NC + lax.axis_index("c")
        base = wid * b_per_w
        pltpu.sync_copy(uid_hbm.at[pl.ds(wid * n_chunks, n_chunks)],
                        idx_v.at[pl.ds(0, n_chunks)])
        pltpu.sync_copy(iid_hbm.at[pl.ds(wid * n_chunks, n_chunks)],
                        idx_v.at[pl.ds(n_chunks, n_chunks)])

        def fire_g(c):
            tab = ut_hbm if c < n_chunks else it_hbm
            return pltpu.async_copy(
                tab.at[idx_v.at[c]],
                bufs.at[pl.ds((c % NBUF) * IDX_CHUNK, IDX_CHUNK)], sem_g)

        def fire_w(c):
            out = ue_hbm if c < n_chunks else ie_hbm
            off = base + (c % n_chunks) * IDX_CHUNK
            return pltpu.async_copy(
                bufs.at[pl.ds((c % NBUF) * IDX_CHUNK, IDX_CHUNK)],
                out.at[pl.ds(off, IDX_CHUNK)], sem_w)

        gd = [None] * n_total
        wd = [None] * n_total
        for c in range(NBUF):
            gd[c] = fire_g(c)
        for c in range(n_total):
            gd[c].wait()
            wd[c] = fire_w(c)
            if c + NBUF < n_total:
                wd[c].wait()
                gd[c + NBUF] = fire_g(c + NBUF)
        for c in range(n_total - NBUF, n_total):
            wd[c].wait()

    return gather(user_emb, item_emb, uid2, iid2)


def _ln_mxu(h, g, b, eps=1e-5):
    n = h.shape[-1]
    avg = jnp.full((n, n), 1.0 / n, jnp.float32)
    mu = jnp.dot(h, avg, preferred_element_type=jnp.float32)
    d = h - mu
    var = jnp.dot(d * d, avg, preferred_element_type=jnp.float32)
    return d * jax.lax.rsqrt(var + eps) * g + b


def _dot_t(a, w):
    return lax.dot_general(a, w, (((1,), (1,)), ((), ())),
                           preferred_element_type=jnp.float32)


def _mlp_body(ue_ref, ie_ref, w1_ref, b1_ref, g1_ref, be1_ref,
              w2_ref, b2_ref, g2_ref, be2_ref, w3_ref, b3_ref,
              w4_ref, b4_ref, w5_ref, b5_ref, bias_ref, out_ref):
    ue = ue_ref[...].astype(jnp.bfloat16)
    ie = ie_ref[...].astype(jnp.bfloat16)
    w1u = w1_ref[:, :EMB].astype(jnp.bfloat16)
    w1i = w1_ref[:, EMB:].astype(jnp.bfloat16)
    ones_col = jnp.full((EMB, 1), 1.0, jnp.bfloat16)
    fm = jnp.dot(ue * ie, ones_col, preferred_element_type=jnp.float32)
    h = lax.dot_general(ue, w1u, (((1,), (1,)), ((), ())),
                        preferred_element_type=jnp.float32)
    h = h + lax.dot_general(ie, w1i, (((1,), (1,)), ((), ())),
                            preferred_element_type=jnp.float32)
    h = jnp.maximum(h + b1_ref[...], 0.0)
    h = _ln_mxu(h, g1_ref[...], be1_ref[...])
    h = jnp.maximum(_dot_t(h, w2_ref[...]) + b2_ref[...], 0.0)
    h = _ln_mxu(h, g2_ref[...], be2_ref[...])
    h = jnp.maximum(_dot_t(h, w3_ref[...]) + b3_ref[...], 0.0)
    h = jnp.maximum(_dot_t(h, w4_ref[...]) + b4_ref[...], 0.0)
    dnn = jnp.dot(h, w5_ref[...], preferred_element_type=jnp.float32) + b5_ref[...]
    t = fm + dnn + bias_ref[...]
    t = t.reshape(t.shape[0] // 128, 128)
    out_ref[...] = jax.nn.sigmoid(t)


def _mlp(ue, ie, *params, block_b=4096):
    B = ue.shape[0]
    grid = (B // block_b,)

    def _full(p):
        return pl.BlockSpec(p.shape, lambda i: (0,) * p.ndim)

    in_specs = [
        pl.BlockSpec((block_b, EMB), lambda i: (i, 0)),
        pl.BlockSpec((block_b, EMB), lambda i: (i, 0)),
    ] + [_full(p) for p in params]

    return pl.pallas_call(
        _mlp_body,
        grid=grid,
        in_specs=in_specs,
        out_specs=pl.BlockSpec((block_b // 128, 128), lambda i: (i, 0)),
        out_shape=jax.ShapeDtypeStruct((B // 128, 128), jnp.float32),
        compiler_params=pltpu.CompilerParams(
            dimension_semantics=("parallel",),
        ),
    )(ue, ie, *params)


def kernel(x, user_emb, item_emb, bias, W1, b1, g1, be1, W2, b2, g2, be2,
           W3, b3, W4, b4, W5, b5):
    uid2 = x[:, 0].astype(jnp.int32).reshape(-1, IDX_CHUNK)
    iid2 = x[:, 1].astype(jnp.int32).reshape(-1, IDX_CHUNK)
    params = (W1, b1.reshape(1, -1), g1.reshape(1, -1), be1.reshape(1, -1),
              W2, b2.reshape(1, -1), g2.reshape(1, -1), be2.reshape(1, -1),
              W3, b3.reshape(1, -1), W4, b4.reshape(1, -1),
              W5.reshape(-1, 1), b5.reshape(1, 1), bias.reshape(1, 1))
    ue, ie = _gather_embeddings(user_emb, item_emb, uid2, iid2)
    return _mlp(ue, ie, *params).reshape(-1)
